# baseline (speedup 1.0000x reference)
import jax
import jax.numpy as jnp
from jax.experimental import pallas as pl

N = 50000
B = 64
H = 4
L = 2


def _ident_body(x_ref, o_ref):
    o_ref[...] = x_ref[...]


def kernel(x_operator, edge_index_calledby, batch_operator, W_in, b_in, Wk, bk, Wq, bq, Wv, bv, a_rel, m_rel, p_rel, Wa, ba, skip, ln_g, ln_b, Wm, bm, Wt, bt):
    C = W_in.shape[1]
    D = C // H
    src = edge_index_calledby[0]
    dst = edge_index_calledby[1]
    h = x_operator @ W_in + b_in
    for _ in range(L):
        k = (h @ Wk + bk).reshape(N, H, D)
        q = (h @ Wq + bq).reshape(N, H, D)
        v = (h @ Wv + bv).reshape(N, H, D)
        k = jnp.einsum('nhd,hde->nhe', k, a_rel)
        v = jnp.einsum('nhd,hde->nhe', v, m_rel)
        alpha = (q[dst] * k[src]).sum(-1) * p_rel / jnp.sqrt(jnp.float32(D))
        amax = jax.ops.segment_max(alpha, dst, num_segments=N)
        amax = jnp.where(jnp.isfinite(amax), amax, 0.0)
        ex = jnp.exp(alpha - amax[dst])
        denom = jax.ops.segment_sum(ex, dst, num_segments=N)
        attn = ex / (denom[dst] + 1e-16)
        out = jax.ops.segment_sum(v[src] * attn[:, :, None], dst, num_segments=N).reshape(N, C)
        out = jax.nn.gelu(out, approximate=False) @ Wa + ba
        beta = jax.nn.sigmoid(skip)
        h = beta * out + (1.0 - beta) * h
        e = jax.nn.elu(h)
        mu = e.mean(-1, keepdims=True)
        var = ((e - mu) ** 2).mean(-1, keepdims=True)
        h = (e - mu) / jnp.sqrt(var + 1e-5) * ln_g + ln_b
    ssum = jax.ops.segment_sum(h, batch_operator, num_segments=B)
    cnt = jax.ops.segment_sum(jnp.ones((N,), h.dtype), batch_operator, num_segments=B)
    emb = ssum / jnp.maximum(cnt, 1.0)[:, None]
    emb = pl.pallas_call(
        _ident_body,
        out_shape=jax.ShapeDtypeStruct(emb.shape, emb.dtype),
    )(emb)
    mem = (emb @ Wm + bm).squeeze()
    tim = (emb @ Wt + bt).squeeze()
    return mem, tim


# speedup vs baseline: 17.8756x; 17.8756x over previous
import functools

import jax
import jax.numpy as jnp
from jax import lax
from jax.experimental import pallas as pl
from jax.experimental.pallas import tpu as pltpu
from jax.experimental.pallas import tpu_sc as plsc

NN = 50000
EE = 800000
CC = 64
HH = 4
DD = 16
BB = 64
LL = 2
CH2 = CC // 2

BN = 2000
NTILES = 16
PER_TILE = EE // NTILES
CH = 32
NCHUNK = PER_TILE // CH
REM = PER_TILE - NCHUNK * CH
ROWW = 40
Z_A = 3128
Z_B = NN - (NTILES - 1) * Z_A

_GRID_N = NN // BN


def _row_spec(width):
    return pl.BlockSpec((BN, width), lambda i: (i, 0))


def _full_spec(shape):
    return pl.BlockSpec(shape, lambda i: tuple(0 for _ in shape))


def _split_store(refs, val):
    r0, r1 = refs
    r0[...] = val[:, 0:CH2]
    r1[...] = val[:, CH2:CC]


def _proj_body(x_ref, Win_ref, bin_ref, Wq_ref, bq_ref, Wk_ref, bk_ref,
               Wv_ref, bv_ref, h_ref, q0, q1, k0, k1, v0, v1):
    h = jnp.dot(x_ref[...], Win_ref[...], preferred_element_type=jnp.float32)
    h = h + bin_ref[...]
    h_ref[...] = h
    _split_store((q0, q1), jnp.dot(h, Wq_ref[...], preferred_element_type=jnp.float32) + bq_ref[...])
    _split_store((k0, k1), jnp.dot(h, Wk_ref[...], preferred_element_type=jnp.float32) + bk_ref[...])
    _split_store((v0, v1), jnp.dot(h, Wv_ref[...], preferred_element_type=jnp.float32) + bv_ref[...])


def _proj_in(x, W_in, b_in, Wq, bq, Wk, bk, Wv, bv):
    f_in = x.shape[1]
    out_shape = ([jax.ShapeDtypeStruct((NN, CC), jnp.float32)]
                 + [jax.ShapeDtypeStruct((NN, CH2), jnp.float32)] * 6)
    return pl.pallas_call(
        _proj_body,
        grid=(_GRID_N,),
        in_specs=[
            _row_spec(f_in),
            _full_spec((f_in, CC)), _full_spec((1, CC)),
            _full_spec((CC, CC)), _full_spec((1, CC)),
            _full_spec((CC, CC)), _full_spec((1, CC)),
            _full_spec((CC, CC)), _full_spec((1, CC)),
        ],
        out_specs=[_row_spec(CC)] + [_row_spec(CH2)] * 6,
        out_shape=out_shape,
    )(x, W_in, b_in, Wq, bq, Wk, bk, Wv, bv)


_SQRT2 = 1.4142135623730951


def _stage_b_math(nm0_ref, nm1_ref, h_ref, S_ref, Wa_ref, ba_ref, beta_ref, g_ref, lb_ref):
    numer = jnp.concatenate([nm0_ref[:, 0:32], nm1_ref[:, 0:32]], axis=1)
    dn = jnp.concatenate([nm0_ref[:, 32:34], nm1_ref[:, 32:34]], axis=1)
    dn64 = jnp.dot(dn, S_ref[...], preferred_element_type=jnp.float32)
    out = numer / (dn64 + 1e-16)
    out = 0.5 * out * (1.0 + lax.erf(out / _SQRT2))
    out = jnp.dot(out, Wa_ref[...], preferred_element_type=jnp.float32) + ba_ref[...]
    beta = beta_ref[...]
    hn = beta * out + (1.0 - beta) * h_ref[...]
    e = jnp.where(hn > 0, hn, jnp.exp(hn) - 1.0)
    mu = jnp.mean(e, axis=1, keepdims=True)
    var = jnp.mean((e - mu) ** 2, axis=1, keepdims=True)
    return (e - mu) * lax.rsqrt(var + 1e-5) * g_ref[...] + lb_ref[...]


def _stageb_proj_body(nm0_ref, nm1_ref, h_ref, S_ref, Wa_ref, ba_ref, beta_ref, g_ref,
                      lb_ref, Wq_ref, bq_ref, Wk_ref, bk_ref, Wv_ref, bv_ref,
                      h_out, q0, q1, k0, k1, v0, v1):
    hln = _stage_b_math(nm0_ref, nm1_ref, h_ref, S_ref, Wa_ref, ba_ref, beta_ref, g_ref, lb_ref)
    h_out[...] = hln
    _split_store((q0, q1), jnp.dot(hln, Wq_ref[...], preferred_element_type=jnp.float32) + bq_ref[...])
    _split_store((k0, k1), jnp.dot(hln, Wk_ref[...], preferred_element_type=jnp.float32) + bk_ref[...])
    _split_store((v0, v1), jnp.dot(hln, Wv_ref[...], preferred_element_type=jnp.float32) + bv_ref[...])


def _stageb_only_body(nm0_ref, nm1_ref, h_ref, S_ref, Wa_ref, ba_ref, beta_ref, g_ref,
                      lb_ref, h_out):
    h_out[...] = _stage_b_math(nm0_ref, nm1_ref, h_ref, S_ref, Wa_ref, ba_ref,
                               beta_ref, g_ref, lb_ref)


_STAGEB_SPECS = [
    _row_spec(ROWW),
    _row_spec(ROWW),
    _row_spec(CC),
    _full_spec((HH, CC)),
    _full_spec((CC, CC)), _full_spec((1, CC)),
    _full_spec((1, 1)),
    _full_spec((1, CC)), _full_spec((1, CC)),
]


def _stage_b_proj(nm0, nm1, h, S, Wa, ba, beta, ln_g, ln_b, Wq, bq, Wk, bk, Wv, bv):
    return pl.pallas_call(
        _stageb_proj_body,
        grid=(_GRID_N,),
        in_specs=_STAGEB_SPECS + [
            _full_spec((CC, CC)), _full_spec((1, CC)),
            _full_spec((CC, CC)), _full_spec((1, CC)),
            _full_spec((CC, CC)), _full_spec((1, CC)),
        ],
        out_specs=[_row_spec(CC)] + [_row_spec(CH2)] * 6,
        out_shape=([jax.ShapeDtypeStruct((NN, CC), jnp.float32)]
                   + [jax.ShapeDtypeStruct((NN, CH2), jnp.float32)] * 6),
    )(nm0, nm1, h, S, Wa, ba, beta, ln_g, ln_b, Wq, bq, Wk, bk, Wv, bv)


def _stage_b_only(nm0, nm1, h, S, Wa, ba, beta, ln_g, ln_b):
    return pl.pallas_call(
        _stageb_only_body,
        grid=(_GRID_N,),
        in_specs=_STAGEB_SPECS,
        out_specs=_row_spec(CC),
        out_shape=jax.ShapeDtypeStruct((NN, CC), jnp.float32),
    )(nm0, nm1, h, S, Wa, ba, beta, ln_g, ln_b)


def _pool_body(h_ref, bo_ref, Wmt_ref, bmt_ref, o_ref, ssum, cnt):
    i = pl.program_id(0)
    h = h_ref[...]
    bo = bo_ref[...]
    oh = (bo == lax.broadcasted_iota(jnp.int32, (BN, BB), 1)).astype(jnp.float32)
    ss = lax.dot_general(oh, h, (((0,), (0,)), ((), ())),
                         preferred_element_type=jnp.float32)
    cn = lax.dot_general(oh, jnp.ones((BN, BB), jnp.float32),
                         (((0,), (0,)), ((), ())),
                         preferred_element_type=jnp.float32)

    @pl.when(i == 0)
    def _():
        ssum[...] = ss
        cnt[...] = cn

    @pl.when(i > 0)
    def _():
        ssum[...] += ss
        cnt[...] += cn

    @pl.when(i == _GRID_N - 1)
    def _():
        emb = ssum[...] / jnp.maximum(cnt[...], 1.0)
        o_ref[...] = jnp.dot(emb, Wmt_ref[...],
                             preferred_element_type=jnp.float32) + bmt_ref[...]


def _pool(h, bo, Wmt, bmt):
    return pl.pallas_call(
        _pool_body,
        grid=(_GRID_N,),
        in_specs=[
            _row_spec(CC),
            _row_spec(1),
            _full_spec((CC, 2)), _full_spec((1, 2)),
        ],
        out_specs=pl.BlockSpec((BB, 2), lambda i: (0, 0)),
        out_shape=jax.ShapeDtypeStruct((BB, 2), jnp.float32),
        scratch_shapes=[
            pltpu.VMEM((BB, BB), jnp.float32),
            pltpu.VMEM((BB, BB), jnp.float32),
        ],
    )(h, bo, Wmt, bmt)


def _sc_edge_body(q0_hbm, q1_hbm, k0_hbm, k1_hbm, v0_hbm, v1_hbm,
                  src_hbm, dst_hbm, zrow_hbm, out0_hbm, out1_hbm,
                  idx_s, idx_d, rdx_s, rdx_d, qbuf, kbuf, vbuf, prod, acc, sem):
    c = lax.axis_index("c")
    t = lax.axis_index("s")

    r0 = t * Z_A

    @pl.when(t < NTILES - 1)
    def _():
        pltpu.sync_copy(zrow_hbm, acc.at[pl.ds(r0, Z_A)])

    @pl.when(t == NTILES - 1)
    def _():
        pltpu.sync_copy(zrow_hbm.at[pl.ds(0, Z_B)], acc.at[pl.ds(r0, Z_B)])

    plsc.subcore_barrier()

    ebase = t * PER_TILE

    def compute_groups(ngroups):
        for g in range(ngroups):
            eidx = lax.iota(jnp.int32, 16) + (g * 16)
            for h in range(2):
                colbase = h * DD
                alpha = jnp.zeros((16,), jnp.float32)
                for d in range(DD):
                    cv16 = jnp.full((16,), colbase + d, jnp.int32)
                    qt = plsc.load_gather(qbuf, [eidx, cv16])
                    kt = plsc.load_gather(kbuf, [eidx, cv16])
                    alpha = alpha + qt * kt
                ex = jnp.exp(alpha)
                plsc.store_scatter(prod, [eidx, jnp.full((16,), 32 + h, jnp.int32)], ex)
                for d in range(DD):
                    cv16 = jnp.full((16,), colbase + d, jnp.int32)
                    vt = plsc.load_gather(vbuf, [eidx, cv16])
                    plsc.store_scatter(prod, [eidx, cv16], vt * ex)

    zero16 = jnp.zeros((16,), jnp.float32)
    for g in range(CH // 16):
        eidx = lax.iota(jnp.int32, 16) + (g * 16)
        for col in range(34, ROWW):
            plsc.store_scatter(prod, [eidx, jnp.full((16,), col, jnp.int32)], zero16)

    def gathers(qh, kh, vh, isr, idr, qb, kb, vb):
        cq = pltpu.async_copy(qh.at[idr], qb, sem)
        ck = pltpu.async_copy(kh.at[isr], kb, sem)
        cv = pltpu.async_copy(vh.at[isr], vb, sem)
        cq.wait()
        ck.wait()
        cv.wait()

    def chunk(j, carry):
        e0 = ebase + j * CH
        pltpu.sync_copy(src_hbm.at[pl.ds(e0, CH)], idx_s)
        pltpu.sync_copy(dst_hbm.at[pl.ds(e0, CH)], idx_d)

        @pl.when(c == 0)
        def _():
            gathers(q0_hbm, k0_hbm, v0_hbm, idx_s, idx_d, qbuf, kbuf, vbuf)

        @pl.when(c == 1)
        def _():
            gathers(q1_hbm, k1_hbm, v1_hbm, idx_s, idx_d, qbuf, kbuf, vbuf)

        compute_groups(CH // 16)
        pltpu.sync_copy(prod, acc.at[idx_d], add=True)
        return carry

    lax.fori_loop(0, NCHUNK, chunk, 0)

    er = ebase + NCHUNK * CH
    pltpu.sync_copy(src_hbm.at[pl.ds(er, REM)], rdx_s)
    pltpu.sync_copy(dst_hbm.at[pl.ds(er, REM)], rdx_d)
    qb_r = qbuf.at[pl.ds(0, REM)]
    kb_r = kbuf.at[pl.ds(0, REM)]
    vb_r = vbuf.at[pl.ds(0, REM)]

    @pl.when(c == 0)
    def _():
        gathers(q0_hbm, k0_hbm, v0_hbm, rdx_s, rdx_d, qb_r, kb_r, vb_r)

    @pl.when(c == 1)
    def _():
        gathers(q1_hbm, k1_hbm, v1_hbm, rdx_s, rdx_d, qb_r, kb_r, vb_r)

    compute_groups(REM // 16)
    pltpu.sync_copy(prod.at[pl.ds(0, REM)], acc.at[rdx_d], add=True)

    plsc.subcore_barrier()

    @pl.when(c == 0)
    def _():
        @pl.when(t < NTILES - 1)
        def _():
            pltpu.sync_copy(acc.at[pl.ds(r0, Z_A)], out0_hbm.at[pl.ds(r0, Z_A)])

        @pl.when(t == NTILES - 1)
        def _():
            pltpu.sync_copy(acc.at[pl.ds(r0, Z_B)], out0_hbm.at[pl.ds(r0, Z_B)])

    @pl.when(c == 1)
    def _():
        @pl.when(t < NTILES - 1)
        def _():
            pltpu.sync_copy(acc.at[pl.ds(r0, Z_A)], out1_hbm.at[pl.ds(r0, Z_A)])

        @pl.when(t == NTILES - 1)
        def _():
            pltpu.sync_copy(acc.at[pl.ds(r0, Z_B)], out1_hbm.at[pl.ds(r0, Z_B)])


@functools.cache
def _build_sc_edge():
    mesh = plsc.VectorSubcoreMesh(core_axis_name="c", subcore_axis_name="s",
                                  num_cores=2, num_subcores=NTILES)
    return functools.partial(
        pl.kernel,
        out_type=[jax.ShapeDtypeStruct((NN, ROWW), jnp.float32),
                  jax.ShapeDtypeStruct((NN, ROWW), jnp.float32)],
        mesh=mesh,
        scratch_types=[
            pltpu.VMEM((CH,), jnp.int32),
            pltpu.VMEM((CH,), jnp.int32),
            pltpu.VMEM((REM,), jnp.int32),
            pltpu.VMEM((REM,), jnp.int32),
            pltpu.VMEM((CH, CH2), jnp.float32),
            pltpu.VMEM((CH, CH2), jnp.float32),
            pltpu.VMEM((CH, CH2), jnp.float32),
            pltpu.VMEM((CH, ROWW), jnp.float32),
            pltpu.VMEM_SHARED((NN, ROWW), jnp.float32),
            pltpu.SemaphoreType.DMA,
        ],
        compiler_params=pltpu.CompilerParams(use_tc_tiling_on_sc=False,
                                             needs_layout_passes=False),
    )(_sc_edge_body)


def _sc_edge(q0, q1, k0, k1, v0, v1, src, dst, zrow):
    return _build_sc_edge()(q0, q1, k0, k1, v0, v1, src, dst, zrow)


def kernel(x_operator, edge_index_calledby, batch_operator, W_in, b_in, Wk, bk,
           Wq, bq, Wv, bv, a_rel, m_rel, p_rel, Wa, ba, skip, ln_g, ln_b, Wm,
           bm, Wt, bt):
    f32 = jnp.float32
    src = edge_index_calledby[0]
    dst = edge_index_calledby[1]

    colscale = jnp.repeat(p_rel / jnp.sqrt(jnp.float32(DD)), DD)
    Wq_s = Wq * colscale[None, :]
    bq_s = (bq * colscale).reshape(1, CC)
    A_blk = jax.scipy.linalg.block_diag(*a_rel)
    M_blk = jax.scipy.linalg.block_diag(*m_rel)
    WkA = Wk @ A_blk
    bkA = (bk @ A_blk).reshape(1, CC)
    WvM = Wv @ M_blk
    bvM = (bv @ M_blk).reshape(1, CC)
    S = jnp.repeat(jnp.eye(HH, dtype=f32), DD, axis=1)
    beta = jax.nn.sigmoid(skip).reshape(1, 1)
    Wmt = jnp.concatenate([Wm, Wt], axis=1)
    bmt = jnp.concatenate([bm, bt]).reshape(1, 2)
    bo = batch_operator.reshape(NN, 1)
    zrow = jnp.zeros((Z_A, ROWW), f32)

    h, q0, q1, k0, k1, v0, v1 = _proj_in(x_operator, W_in, b_in.reshape(1, CC),
                                         Wq_s, bq_s, WkA, bkA, WvM, bvM)
    for layer in range(LL):
        nm0, nm1 = _sc_edge(q0, q1, k0, k1, v0, v1, src, dst, zrow)
        if layer < LL - 1:
            h, q0, q1, k0, k1, v0, v1 = _stage_b_proj(
                nm0, nm1, h, S, Wa, ba.reshape(1, CC), beta,
                ln_g.reshape(1, CC), ln_b.reshape(1, CC),
                Wq_s, bq_s, WkA, bkA, WvM, bvM)
        else:
            h = _stage_b_only(nm0, nm1, h, S, Wa, ba.reshape(1, CC), beta,
                              ln_g.reshape(1, CC), ln_b.reshape(1, CC))

    out2 = _pool(h, bo, Wmt, bmt)
    return out2[:, 0], out2[:, 1]


# speedup vs baseline: 21.5126x; 1.2035x over previous
import functools

import jax
import jax.numpy as jnp
from jax import lax
from jax.experimental import pallas as pl
from jax.experimental.pallas import tpu as pltpu
from jax.experimental.pallas import tpu_sc as plsc

NN = 50000
EE = 800000
CC = 64
HH = 4
DD = 16
BB = 64
LL = 2
CH2 = CC // 2

BN = 2000
NTILES = 16
PER_TILE = EE // NTILES
CH = 64
NCHUNK = PER_TILE // CH
NPAIR = (NCHUNK - 1) // 2
REM = PER_TILE - NCHUNK * CH
ND = NN // 4
Z_A = 3128
Z_B = NN - (NTILES - 1) * Z_A
D_A = 784
D_B = ND - (NTILES - 1) * D_A

_GRID_N = NN // BN


def _row_spec(width):
    return pl.BlockSpec((BN, width), lambda i: (i, 0))


def _full_spec(shape):
    return pl.BlockSpec(shape, lambda i: tuple(0 for _ in shape))


def _proj_stores(h, Wq_ref, bq_ref, Wk_ref, bk_ref, Wv_ref, bv_ref,
                 q_out, kv_out):
    q = jnp.dot(h, Wq_ref[...], preferred_element_type=jnp.float32) + bq_ref[...]
    k = jnp.dot(h, Wk_ref[...], preferred_element_type=jnp.float32) + bk_ref[...]
    v = jnp.dot(h, Wv_ref[...], preferred_element_type=jnp.float32) + bv_ref[...]
    q_out[...] = q
    kv_out[...] = jnp.concatenate([k[:, 0:CH2], v[:, 0:CH2],
                                   k[:, CH2:CC], v[:, CH2:CC]], axis=1)


def _proj_body(x_ref, Win_ref, bin_ref, Wq_ref, bq_ref, Wk_ref, bk_ref,
               Wv_ref, bv_ref, h_ref, q_out, kv_out):
    h = jnp.dot(x_ref[...], Win_ref[...], preferred_element_type=jnp.float32)
    h = h + bin_ref[...]
    h_ref[...] = h
    _proj_stores(h, Wq_ref, bq_ref, Wk_ref, bk_ref, Wv_ref, bv_ref,
                 q_out, kv_out)


_PROJ_OUT_SHAPES = [jax.ShapeDtypeStruct((NN, CC), jnp.float32),
                    jax.ShapeDtypeStruct((NN, CC), jnp.float32),
                    jax.ShapeDtypeStruct((NN, 2 * CC), jnp.float32)]
_PROJ_OUT_SPECS = [_row_spec(CC), _row_spec(CC), _row_spec(2 * CC)]


def _proj_in(x, W_in, b_in, Wq, bq, Wk, bk, Wv, bv):
    f_in = x.shape[1]
    return pl.pallas_call(
        _proj_body,
        grid=(_GRID_N,),
        in_specs=[
            _row_spec(f_in),
            _full_spec((f_in, CC)), _full_spec((1, CC)),
            _full_spec((CC, CC)), _full_spec((1, CC)),
            _full_spec((CC, CC)), _full_spec((1, CC)),
            _full_spec((CC, CC)), _full_spec((1, CC)),
        ],
        out_specs=_PROJ_OUT_SPECS,
        out_shape=_PROJ_OUT_SHAPES,
    )(x, W_in, b_in, Wq, bq, Wk, bk, Wv, bv)


_SQRT2 = 1.4142135623730951


def _stage_b_math(nm0_ref, nm1_ref, dn_ref, h_ref, S_ref, Wa_ref, ba_ref,
                  beta_ref, g_ref, lb_ref):
    numer = jnp.concatenate([nm0_ref[...], nm1_ref[...]], axis=1)
    dn64 = jnp.dot(dn_ref[...], S_ref[...], preferred_element_type=jnp.float32)
    out = numer / (dn64 + 1e-16)
    out = 0.5 * out * (1.0 + lax.erf(out / _SQRT2))
    out = jnp.dot(out, Wa_ref[...], preferred_element_type=jnp.float32) + ba_ref[...]
    beta = beta_ref[...]
    hn = beta * out + (1.0 - beta) * h_ref[...]
    e = jnp.where(hn > 0, hn, jnp.exp(hn) - 1.0)
    mu = jnp.mean(e, axis=1, keepdims=True)
    var = jnp.mean((e - mu) ** 2, axis=1, keepdims=True)
    return (e - mu) * lax.rsqrt(var + 1e-5) * g_ref[...] + lb_ref[...]


def _stageb_proj_body(nm0_ref, nm1_ref, dn_ref, h_ref, S_ref, Wa_ref, ba_ref,
                      beta_ref, g_ref, lb_ref, Wq_ref, bq_ref, Wk_ref, bk_ref,
                      Wv_ref, bv_ref, h_out, q_out, kv_out):
    hln = _stage_b_math(nm0_ref, nm1_ref, dn_ref, h_ref, S_ref, Wa_ref, ba_ref,
                        beta_ref, g_ref, lb_ref)
    h_out[...] = hln
    _proj_stores(hln, Wq_ref, bq_ref, Wk_ref, bk_ref, Wv_ref, bv_ref,
                 q_out, kv_out)


def _stageb_only_body(nm0_ref, nm1_ref, dn_ref, h_ref, S_ref, Wa_ref, ba_ref,
                      beta_ref, g_ref, lb_ref, h_out):
    h_out[...] = _stage_b_math(nm0_ref, nm1_ref, dn_ref, h_ref, S_ref, Wa_ref,
                               ba_ref, beta_ref, g_ref, lb_ref)


_STAGEB_SPECS = [
    _row_spec(CH2),
    _row_spec(CH2),
    _row_spec(HH),
    _row_spec(CC),
    _full_spec((HH, CC)),
    _full_spec((CC, CC)), _full_spec((1, CC)),
    _full_spec((1, 1)),
    _full_spec((1, CC)), _full_spec((1, CC)),
]


def _stage_b_proj(nm0, nm1, dn, h, S, Wa, ba, beta, ln_g, ln_b,
                  Wq, bq, Wk, bk, Wv, bv):
    return pl.pallas_call(
        _stageb_proj_body,
        grid=(_GRID_N,),
        in_specs=_STAGEB_SPECS + [
            _full_spec((CC, CC)), _full_spec((1, CC)),
            _full_spec((CC, CC)), _full_spec((1, CC)),
            _full_spec((CC, CC)), _full_spec((1, CC)),
        ],
        out_specs=_PROJ_OUT_SPECS,
        out_shape=_PROJ_OUT_SHAPES,
    )(nm0, nm1, dn, h, S, Wa, ba, beta, ln_g, ln_b, Wq, bq, Wk, bk, Wv, bv)


def _stage_b_only(nm0, nm1, dn, h, S, Wa, ba, beta, ln_g, ln_b):
    return pl.pallas_call(
        _stageb_only_body,
        grid=(_GRID_N,),
        in_specs=_STAGEB_SPECS,
        out_specs=_row_spec(CC),
        out_shape=jax.ShapeDtypeStruct((NN, CC), jnp.float32),
    )(nm0, nm1, dn, h, S, Wa, ba, beta, ln_g, ln_b)


def _pool_body(h_ref, bo_ref, Wmt_ref, bmt_ref, o_ref, ssum, cnt):
    i = pl.program_id(0)
    h = h_ref[...]
    bo = bo_ref[...]
    oh = (bo == lax.broadcasted_iota(jnp.int32, (BN, BB), 1)).astype(jnp.float32)
    ss = lax.dot_general(oh, h, (((0,), (0,)), ((), ())),
                         preferred_element_type=jnp.float32)
    cn = lax.dot_general(oh, jnp.ones((BN, BB), jnp.float32),
                         (((0,), (0,)), ((), ())),
                         preferred_element_type=jnp.float32)

    @pl.when(i == 0)
    def _():
        ssum[...] = ss
        cnt[...] = cn

    @pl.when(i > 0)
    def _():
        ssum[...] += ss
        cnt[...] += cn

    @pl.when(i == _GRID_N - 1)
    def _():
        emb = ssum[...] / jnp.maximum(cnt[...], 1.0)
        o_ref[...] = jnp.dot(emb, Wmt_ref[...],
                             preferred_element_type=jnp.float32) + bmt_ref[...]


def _pool(h, bo, Wmt, bmt):
    return pl.pallas_call(
        _pool_body,
        grid=(_GRID_N,),
        in_specs=[
            _row_spec(CC),
            _row_spec(1),
            _full_spec((CC, 2)), _full_spec((1, 2)),
        ],
        out_specs=pl.BlockSpec((BB, 2), lambda i: (0, 0)),
        out_shape=jax.ShapeDtypeStruct((BB, 2), jnp.float32),
        scratch_shapes=[
            pltpu.VMEM((BB, BB), jnp.float32),
            pltpu.VMEM((BB, BB), jnp.float32),
        ],
    )(h, bo, Wmt, bmt)


def _sc_edge_body(q_hbm, kv_hbm, src_hbm, dst_hbm,
                  zn_hbm, zd_hbm, nm0_hbm, nm1_hbm, dn0_hbm, dn1_hbm,
                  is_r, id_a, id_b, ic_sa, ic_sb, ic_da, ic_db, id4,
                  jr_d, jc_s, jc_d, jd4,
                  qb_a, qb_b, kvb_a, kvb_b, vprod, dprod, nacc, dacc,
                  sem_a, sem_b):
    c = lax.axis_index("c")
    t = lax.axis_index("s")

    r0 = t * Z_A
    s0 = t * D_A

    @pl.when(t < NTILES - 1)
    def _():
        pltpu.sync_copy(zn_hbm, nacc.at[pl.ds(r0, Z_A)])
        pltpu.sync_copy(zd_hbm, dacc.at[pl.ds(s0, D_A)])

    @pl.when(t == NTILES - 1)
    def _():
        pltpu.sync_copy(zn_hbm.at[pl.ds(0, Z_B)], nacc.at[pl.ds(r0, Z_B)])
        pltpu.sync_copy(zd_hbm.at[pl.ds(0, D_B)], dacc.at[pl.ds(s0, D_B)])

    plsc.subcore_barrier()

    ebase = t * PER_TILE
    zero16 = jnp.zeros((16,), jnp.float32)

    def fire(j, id_x, ic_s, ic_d, qb, kvb, sem):
        e0 = ebase + j * CH
        pltpu.sync_copy(src_hbm.at[pl.ds(e0, CH)], is_r)
        pltpu.sync_copy(dst_hbm.at[pl.ds(e0, CH)], id_x)
        for g in range(CH // 16):
            sl = pl.ds(g * 16, 16)
            ic_s[sl] = lax.shift_left(is_r[sl], 1) + c
            ic_d[sl] = lax.shift_left(id_x[sl], 1) + c
        pltpu.async_copy(q_hbm.at[ic_d], qb, sem)
        pltpu.async_copy(kv_hbm.at[ic_s], kvb, sem)

    def wait(ic_s, ic_d, qb, kvb, sem):
        pltpu.make_async_copy(q_hbm.at[ic_d], qb, sem).wait()
        pltpu.make_async_copy(kv_hbm.at[ic_s], kvb, sem).wait()

    def group_math(id_x, qb, kvb, g):
        eidx = lax.iota(jnp.int32, 16) + (g * 16)
        dstv = id_x[pl.ds(g * 16, 16)]
        id4[pl.ds(g * 16, 16)] = lax.shift_right_logical(dstv, 2)
        colb = lax.shift_left(jnp.bitwise_and(dstv, 3), 1)
        for col8 in range(8):
            plsc.store_scatter(dprod, [eidx, jnp.full((16,), col8, jnp.int32)], zero16)
        for h in range(2):
            alpha = jnp.zeros((16,), jnp.float32)
            for d in range(DD):
                cv16 = jnp.full((16,), h * DD + d, jnp.int32)
                qt = plsc.load_gather(qb, [eidx, cv16])
                kt = plsc.load_gather(kvb, [eidx, cv16])
                alpha = alpha + qt * kt
            ex = jnp.exp(alpha)
            plsc.store_scatter(dprod, [eidx, colb + h], ex)
            for d in range(DD):
                vt = plsc.load_gather(kvb, [eidx, jnp.full((16,), CH2 + h * DD + d, jnp.int32)])
                plsc.store_scatter(vprod, [eidx, jnp.full((16,), h * DD + d, jnp.int32)], vt * ex)

    def compute_scatter(id_x, qb, kvb):
        for g in range(CH // 16):
            group_math(id_x, qb, kvb, g)
        pltpu.sync_copy(vprod, nacc.at[id_x], add=True)
        pltpu.sync_copy(dprod, dacc.at[id4], add=True)

    fire(0, id_a, ic_sa, ic_da, qb_a, kvb_a, sem_a)

    def pair(i, carry):
        j = i * 2
        fire(j + 1, id_b, ic_sb, ic_db, qb_b, kvb_b, sem_b)
        wait(ic_sa, ic_da, qb_a, kvb_a, sem_a)
        compute_scatter(id_a, qb_a, kvb_a)
        fire(j + 2, id_a, ic_sa, ic_da, qb_a, kvb_a, sem_a)
        wait(ic_sb, ic_db, qb_b, kvb_b, sem_b)
        compute_scatter(id_b, qb_b, kvb_b)
        return carry

    lax.fori_loop(0, NPAIR, pair, 0)
    wait(ic_sa, ic_da, qb_a, kvb_a, sem_a)
    compute_scatter(id_a, qb_a, kvb_a)

    er = ebase + NCHUNK * CH
    pltpu.sync_copy(src_hbm.at[pl.ds(er, REM)], jc_s)
    pltpu.sync_copy(dst_hbm.at[pl.ds(er, REM)], jr_d)
    jc_s[...] = lax.shift_left(jc_s[...], 1) + c
    jc_d[...] = lax.shift_left(jr_d[...], 1) + c
    qr = qb_a.at[pl.ds(0, REM)]
    kvr = kvb_a.at[pl.ds(0, REM)]
    pltpu.async_copy(q_hbm.at[jc_d], qr, sem_a)
    pltpu.async_copy(kv_hbm.at[jc_s], kvr, sem_a)
    pltpu.make_async_copy(q_hbm.at[jc_d], qr, sem_a).wait()
    pltpu.make_async_copy(kv_hbm.at[jc_s], kvr, sem_a).wait()

    eidx = lax.iota(jnp.int32, 16)
    dstv = jr_d[...]
    jd4[...] = lax.shift_right_logical(dstv, 2)
    colb = lax.shift_left(jnp.bitwise_and(dstv, 3), 1)
    for col8 in range(8):
        plsc.store_scatter(dprod, [eidx, jnp.full((16,), col8, jnp.int32)], zero16)
    for h in range(2):
        alpha = jnp.zeros((16,), jnp.float32)
        for d in range(DD):
            cv16 = jnp.full((16,), h * DD + d, jnp.int32)
            qt = plsc.load_gather(qb_a, [eidx, cv16])
            kt = plsc.load_gather(kvb_a, [eidx, cv16])
            alpha = alpha + qt * kt
        ex = jnp.exp(alpha)
        plsc.store_scatter(dprod, [eidx, colb + h], ex)
        for d in range(DD):
            vt = plsc.load_gather(kvb_a, [eidx, jnp.full((16,), CH2 + h * DD + d, jnp.int32)])
            plsc.store_scatter(vprod, [eidx, jnp.full((16,), h * DD + d, jnp.int32)], vt * ex)
    pltpu.sync_copy(vprod.at[pl.ds(0, REM)], nacc.at[jr_d], add=True)
    pltpu.sync_copy(dprod.at[pl.ds(0, REM)], dacc.at[jd4], add=True)

    plsc.subcore_barrier()

    def writeout(nm_hbm, dn_hbm):
        @pl.when(t < NTILES - 1)
        def _():
            pltpu.sync_copy(nacc.at[pl.ds(r0, Z_A)], nm_hbm.at[pl.ds(r0, Z_A)])
            pltpu.sync_copy(dacc.at[pl.ds(s0, D_A)], dn_hbm.at[pl.ds(s0, D_A)])

        @pl.when(t == NTILES - 1)
        def _():
            pltpu.sync_copy(nacc.at[pl.ds(r0, Z_B)], nm_hbm.at[pl.ds(r0, Z_B)])
            pltpu.sync_copy(dacc.at[pl.ds(s0, D_B)], dn_hbm.at[pl.ds(s0, D_B)])

    @pl.when(c == 0)
    def _():
        writeout(nm0_hbm, dn0_hbm)

    @pl.when(c == 1)
    def _():
        writeout(nm1_hbm, dn1_hbm)


@functools.cache
def _build_sc_edge():
    mesh = plsc.VectorSubcoreMesh(core_axis_name="c", subcore_axis_name="s",
                                  num_cores=2, num_subcores=NTILES)
    return functools.partial(
        pl.kernel,
        out_type=[jax.ShapeDtypeStruct((NN, CH2), jnp.float32),
                  jax.ShapeDtypeStruct((NN, CH2), jnp.float32),
                  jax.ShapeDtypeStruct((ND, 8), jnp.float32),
                  jax.ShapeDtypeStruct((ND, 8), jnp.float32)],
        mesh=mesh,
        scratch_types=(
            [pltpu.VMEM((CH,), jnp.int32)] * 8
            + [pltpu.VMEM((REM,), jnp.int32)] * 4
            + [
                pltpu.VMEM((CH, CH2), jnp.float32),
                pltpu.VMEM((CH, CH2), jnp.float32),
                pltpu.VMEM((CH, CC), jnp.float32),
                pltpu.VMEM((CH, CC), jnp.float32),
                pltpu.VMEM((CH, CH2), jnp.float32),
                pltpu.VMEM((CH, 8), jnp.float32),
                pltpu.VMEM_SHARED((NN, CH2), jnp.float32),
                pltpu.VMEM_SHARED((ND, 8), jnp.float32),
                pltpu.SemaphoreType.DMA,
                pltpu.SemaphoreType.DMA,
            ]
        ),
        compiler_params=pltpu.CompilerParams(use_tc_tiling_on_sc=False,
                                             needs_layout_passes=False),
    )(_sc_edge_body)


def _sc_edge(qi, kvi, src, dst, zn, zd):
    return _build_sc_edge()(qi, kvi, src, dst, zn, zd)


def kernel(x_operator, edge_index_calledby, batch_operator, W_in, b_in, Wk, bk,
           Wq, bq, Wv, bv, a_rel, m_rel, p_rel, Wa, ba, skip, ln_g, ln_b, Wm,
           bm, Wt, bt):
    f32 = jnp.float32
    src = edge_index_calledby[0]
    dst = edge_index_calledby[1]

    colscale = jnp.repeat(p_rel / jnp.sqrt(jnp.float32(DD)), DD)
    Wq_s = Wq * colscale[None, :]
    bq_s = (bq * colscale).reshape(1, CC)
    A_blk = jax.scipy.linalg.block_diag(*a_rel)
    M_blk = jax.scipy.linalg.block_diag(*m_rel)
    WkA = Wk @ A_blk
    bkA = (bk @ A_blk).reshape(1, CC)
    WvM = Wv @ M_blk
    bvM = (bv @ M_blk).reshape(1, CC)
    S = jnp.repeat(jnp.eye(HH, dtype=f32), DD, axis=1)
    beta = jax.nn.sigmoid(skip).reshape(1, 1)
    Wmt = jnp.concatenate([Wm, Wt], axis=1)
    bmt = jnp.concatenate([bm, bt]).reshape(1, 2)
    bo = batch_operator.reshape(NN, 1)
    zn = jnp.zeros((Z_A, CH2), f32)
    zd = jnp.zeros((D_A, 8), f32)

    h, q, kvf = _proj_in(x_operator, W_in, b_in.reshape(1, CC),
                         Wq_s, bq_s, WkA, bkA, WvM, bvM)
    for layer in range(LL):
        nm0, nm1, dp0, dp1 = _sc_edge(q.reshape(2 * NN, CH2),
                                      kvf.reshape(2 * NN, CC),
                                      src, dst, zn, zd)
        dn = jnp.concatenate([dp0.reshape(NN, 2), dp1.reshape(NN, 2)], axis=1)
        if layer < LL - 1:
            h, q, kvf = _stage_b_proj(
                nm0, nm1, dn, h, S, Wa, ba.reshape(1, CC), beta,
                ln_g.reshape(1, CC), ln_b.reshape(1, CC),
                Wq_s, bq_s, WkA, bkA, WvM, bvM)
        else:
            h = _stage_b_only(nm0, nm1, dn, h, S, Wa, ba.reshape(1, CC), beta,
                              ln_g.reshape(1, CC), ln_b.reshape(1, CC))

    out2 = _pool(h, bo, Wmt, bmt)
    return out2[:, 0], out2[:, 1]


# speedup vs baseline: 24.3861x; 1.1336x over previous
import functools

import jax
import jax.numpy as jnp
from jax import lax
from jax.experimental import pallas as pl
from jax.experimental.pallas import tpu as pltpu
from jax.experimental.pallas import tpu_sc as plsc

NN = 50000
EE = 800000
CC = 64
HH = 4
DD = 16
BB = 64
LL = 2
CH2 = CC // 2

BN = 2000
NTILES = 16
PER_TILE = EE // NTILES
CH = 64
NCHUNK = PER_TILE // CH
NPAIR = (NCHUNK - 1) // 2
REM = PER_TILE - NCHUNK * CH
BK = 16
BKE = BK * CH
EPAD = EE + BKE
ND = NN // 4
Z_A = 3128
Z_B = NN - (NTILES - 1) * Z_A
D_A = 784
D_B = ND - (NTILES - 1) * D_A

_GRID_N = NN // BN


def _row_spec(width):
    return pl.BlockSpec((BN, width), lambda i: (i, 0))


def _full_spec(shape):
    return pl.BlockSpec(shape, lambda i: tuple(0 for _ in shape))


def _proj_stores(h, Wq_ref, bq_ref, Wk_ref, bk_ref, Wv_ref, bv_ref,
                 q_out, kv_out):
    q = jnp.dot(h, Wq_ref[...], preferred_element_type=jnp.float32) + bq_ref[...]
    k = jnp.dot(h, Wk_ref[...], preferred_element_type=jnp.float32) + bk_ref[...]
    v = jnp.dot(h, Wv_ref[...], preferred_element_type=jnp.float32) + bv_ref[...]
    q_out[...] = q
    kv_out[...] = jnp.concatenate([k[:, 0:CH2], v[:, 0:CH2],
                                   k[:, CH2:CC], v[:, CH2:CC]], axis=1)


def _proj_body(x_ref, Win_ref, bin_ref, Wq_ref, bq_ref, Wk_ref, bk_ref,
               Wv_ref, bv_ref, h_ref, q_out, kv_out):
    h = jnp.dot(x_ref[...], Win_ref[...], preferred_element_type=jnp.float32)
    h = h + bin_ref[...]
    h_ref[...] = h
    _proj_stores(h, Wq_ref, bq_ref, Wk_ref, bk_ref, Wv_ref, bv_ref,
                 q_out, kv_out)


_PROJ_OUT_SHAPES = [jax.ShapeDtypeStruct((NN, CC), jnp.float32),
                    jax.ShapeDtypeStruct((NN, CC), jnp.float32),
                    jax.ShapeDtypeStruct((NN, 2 * CC), jnp.float32)]
_PROJ_OUT_SPECS = [_row_spec(CC), _row_spec(CC), _row_spec(2 * CC)]


def _proj_in(x, W_in, b_in, Wq, bq, Wk, bk, Wv, bv):
    f_in = x.shape[1]
    return pl.pallas_call(
        _proj_body,
        grid=(_GRID_N,),
        in_specs=[
            _row_spec(f_in),
            _full_spec((f_in, CC)), _full_spec((1, CC)),
            _full_spec((CC, CC)), _full_spec((1, CC)),
            _full_spec((CC, CC)), _full_spec((1, CC)),
            _full_spec((CC, CC)), _full_spec((1, CC)),
        ],
        out_specs=_PROJ_OUT_SPECS,
        out_shape=_PROJ_OUT_SHAPES,
    )(x, W_in, b_in, Wq, bq, Wk, bk, Wv, bv)


_SQRT2 = 1.4142135623730951


def _stage_b_math(nm0_ref, nm1_ref, dn_ref, h_ref, S_ref, Wa_ref, ba_ref,
                  beta_ref, g_ref, lb_ref):
    numer = jnp.concatenate([nm0_ref[...], nm1_ref[...]], axis=1)
    dn64 = jnp.dot(dn_ref[...], S_ref[...], preferred_element_type=jnp.float32)
    out = numer / (dn64 + 1e-16)
    out = 0.5 * out * (1.0 + lax.erf(out / _SQRT2))
    out = jnp.dot(out, Wa_ref[...], preferred_element_type=jnp.float32) + ba_ref[...]
    beta = beta_ref[...]
    hn = beta * out + (1.0 - beta) * h_ref[...]
    e = jnp.where(hn > 0, hn, jnp.exp(hn) - 1.0)
    mu = jnp.mean(e, axis=1, keepdims=True)
    var = jnp.mean((e - mu) ** 2, axis=1, keepdims=True)
    return (e - mu) * lax.rsqrt(var + 1e-5) * g_ref[...] + lb_ref[...]


def _stageb_proj_body(nm0_ref, nm1_ref, dn_ref, h_ref, S_ref, Wa_ref, ba_ref,
                      beta_ref, g_ref, lb_ref, Wq_ref, bq_ref, Wk_ref, bk_ref,
                      Wv_ref, bv_ref, h_out, q_out, kv_out):
    hln = _stage_b_math(nm0_ref, nm1_ref, dn_ref, h_ref, S_ref, Wa_ref, ba_ref,
                        beta_ref, g_ref, lb_ref)
    h_out[...] = hln
    _proj_stores(hln, Wq_ref, bq_ref, Wk_ref, bk_ref, Wv_ref, bv_ref,
                 q_out, kv_out)


def _stageb_only_body(nm0_ref, nm1_ref, dn_ref, h_ref, S_ref, Wa_ref, ba_ref,
                      beta_ref, g_ref, lb_ref, h_out):
    h_out[...] = _stage_b_math(nm0_ref, nm1_ref, dn_ref, h_ref, S_ref, Wa_ref,
                               ba_ref, beta_ref, g_ref, lb_ref)


_STAGEB_SPECS = [
    _row_spec(CH2),
    _row_spec(CH2),
    _row_spec(HH),
    _row_spec(CC),
    _full_spec((HH, CC)),
    _full_spec((CC, CC)), _full_spec((1, CC)),
    _full_spec((1, 1)),
    _full_spec((1, CC)), _full_spec((1, CC)),
]


def _stage_b_proj(nm0, nm1, dn, h, S, Wa, ba, beta, ln_g, ln_b,
                  Wq, bq, Wk, bk, Wv, bv):
    return pl.pallas_call(
        _stageb_proj_body,
        grid=(_GRID_N,),
        in_specs=_STAGEB_SPECS + [
            _full_spec((CC, CC)), _full_spec((1, CC)),
            _full_spec((CC, CC)), _full_spec((1, CC)),
            _full_spec((CC, CC)), _full_spec((1, CC)),
        ],
        out_specs=_PROJ_OUT_SPECS,
        out_shape=_PROJ_OUT_SHAPES,
    )(nm0, nm1, dn, h, S, Wa, ba, beta, ln_g, ln_b, Wq, bq, Wk, bk, Wv, bv)


def _stage_b_only(nm0, nm1, dn, h, S, Wa, ba, beta, ln_g, ln_b):
    return pl.pallas_call(
        _stageb_only_body,
        grid=(_GRID_N,),
        in_specs=_STAGEB_SPECS,
        out_specs=_row_spec(CC),
        out_shape=jax.ShapeDtypeStruct((NN, CC), jnp.float32),
    )(nm0, nm1, dn, h, S, Wa, ba, beta, ln_g, ln_b)


def _pool_body(h_ref, bo_ref, Wmt_ref, bmt_ref, o_ref, ssum, cnt):
    i = pl.program_id(0)
    h = h_ref[...]
    bo = bo_ref[...]
    oh = (bo == lax.broadcasted_iota(jnp.int32, (BN, BB), 1)).astype(jnp.float32)
    ss = lax.dot_general(oh, h, (((0,), (0,)), ((), ())),
                         preferred_element_type=jnp.float32)
    cn = lax.dot_general(oh, jnp.ones((BN, BB), jnp.float32),
                         (((0,), (0,)), ((), ())),
                         preferred_element_type=jnp.float32)

    @pl.when(i == 0)
    def _():
        ssum[...] = ss
        cnt[...] = cn

    @pl.when(i > 0)
    def _():
        ssum[...] += ss
        cnt[...] += cn

    @pl.when(i == _GRID_N - 1)
    def _():
        emb = ssum[...] / jnp.maximum(cnt[...], 1.0)
        o_ref[...] = jnp.dot(emb, Wmt_ref[...],
                             preferred_element_type=jnp.float32) + bmt_ref[...]


def _pool(h, bo, Wmt, bmt):
    return pl.pallas_call(
        _pool_body,
        grid=(_GRID_N,),
        in_specs=[
            _row_spec(CC),
            _row_spec(1),
            _full_spec((CC, 2)), _full_spec((1, 2)),
        ],
        out_specs=pl.BlockSpec((BB, 2), lambda i: (0, 0)),
        out_shape=jax.ShapeDtypeStruct((BB, 2), jnp.float32),
        scratch_shapes=[
            pltpu.VMEM((BB, BB), jnp.float32),
            pltpu.VMEM((BB, BB), jnp.float32),
        ],
    )(h, bo, Wmt, bmt)


def _sc_edge_body(q_hbm, kv_hbm, src_hbm, dst_hbm,
                  zn_hbm, zd_hbm, nm0_hbm, nm1_hbm, dn0_hbm, dn1_hbm,
                  isb, idb, id_a, id_b, ic_sa, ic_sb, ic_da, ic_db, id4,
                  jr_d, jc_s, jc_d, jd4,
                  qb_a, qb_b, kvb_a, kvb_b, vprod, dprod, nacc, dacc,
                  sem_a, sem_b):
    c = lax.axis_index("c")
    t = lax.axis_index("s")

    r0 = t * Z_A
    s0 = t * D_A

    @pl.when(t < NTILES - 1)
    def _():
        pltpu.sync_copy(zn_hbm, nacc.at[pl.ds(r0, Z_A)])
        pltpu.sync_copy(zd_hbm, dacc.at[pl.ds(s0, D_A)])

    @pl.when(t == NTILES - 1)
    def _():
        pltpu.sync_copy(zn_hbm.at[pl.ds(0, Z_B)], nacc.at[pl.ds(r0, Z_B)])
        pltpu.sync_copy(zd_hbm.at[pl.ds(0, D_B)], dacc.at[pl.ds(s0, D_B)])

    plsc.subcore_barrier()

    ebase = t * PER_TILE
    zero16 = jnp.zeros((16,), jnp.float32)

    def fire(j, id_x, ic_s, ic_d, qb, kvb, sem):
        boff = jnp.bitwise_and(j, BK - 1) * CH

        @pl.when(boff == 0)
        def _():
            e0 = ebase + j * CH
            pltpu.sync_copy(src_hbm.at[pl.ds(e0, BKE)], isb)
            pltpu.sync_copy(dst_hbm.at[pl.ds(e0, BKE)], idb)

        for g in range(CH // 16):
            sl = pl.ds(g * 16, 16)
            bl = pl.ds(boff + g * 16, 16)
            sv = isb[bl]
            dv = idb[bl]
            id_x[sl] = dv
            ic_s[sl] = lax.shift_left(sv, 1) + c
            ic_d[sl] = lax.shift_left(dv, 1) + c
        pltpu.async_copy(q_hbm.at[ic_d], qb, sem)
        pltpu.async_copy(kv_hbm.at[ic_s], kvb, sem)

    def wait(ic_s, ic_d, qb, kvb, sem):
        pltpu.make_async_copy(q_hbm.at[ic_d], qb, sem).wait()
        pltpu.make_async_copy(kv_hbm.at[ic_s], kvb, sem).wait()

    def group_math(id_x, qb, kvb, g):
        eidx = lax.iota(jnp.int32, 16) + (g * 16)
        dstv = id_x[pl.ds(g * 16, 16)]
        id4[pl.ds(g * 16, 16)] = lax.shift_right_logical(dstv, 2)
        colb = lax.shift_left(jnp.bitwise_and(dstv, 3), 1)
        for col8 in range(8):
            plsc.store_scatter(dprod, [eidx, jnp.full((16,), col8, jnp.int32)], zero16)
        for h in range(2):
            alpha = jnp.zeros((16,), jnp.float32)
            for d in range(DD):
                cv16 = jnp.full((16,), h * DD + d, jnp.int32)
                qt = plsc.load_gather(qb, [eidx, cv16])
                kt = plsc.load_gather(kvb, [eidx, cv16])
                alpha = alpha + qt * kt
            ex = jnp.exp(alpha)
            plsc.store_scatter(dprod, [eidx, colb + h], ex)
            for d in range(DD):
                vt = plsc.load_gather(kvb, [eidx, jnp.full((16,), CH2 + h * DD + d, jnp.int32)])
                plsc.store_scatter(vprod, [eidx, jnp.full((16,), h * DD + d, jnp.int32)], vt * ex)

    def compute_scatter(id_x, qb, kvb):
        for g in range(CH // 16):
            group_math(id_x, qb, kvb, g)
        pltpu.sync_copy(vprod, nacc.at[id_x], add=True)
        pltpu.sync_copy(dprod, dacc.at[id4], add=True)

    fire(0, id_a, ic_sa, ic_da, qb_a, kvb_a, sem_a)

    def pair(i, carry):
        j = i * 2
        fire(j + 1, id_b, ic_sb, ic_db, qb_b, kvb_b, sem_b)
        wait(ic_sa, ic_da, qb_a, kvb_a, sem_a)
        compute_scatter(id_a, qb_a, kvb_a)
        fire(j + 2, id_a, ic_sa, ic_da, qb_a, kvb_a, sem_a)
        wait(ic_sb, ic_db, qb_b, kvb_b, sem_b)
        compute_scatter(id_b, qb_b, kvb_b)
        return carry

    lax.fori_loop(0, NPAIR, pair, 0)
    wait(ic_sa, ic_da, qb_a, kvb_a, sem_a)
    compute_scatter(id_a, qb_a, kvb_a)

    er = ebase + NCHUNK * CH
    pltpu.sync_copy(src_hbm.at[pl.ds(er, REM)], jc_s)
    pltpu.sync_copy(dst_hbm.at[pl.ds(er, REM)], jr_d)
    jc_s[...] = lax.shift_left(jc_s[...], 1) + c
    jc_d[...] = lax.shift_left(jr_d[...], 1) + c
    qr = qb_a.at[pl.ds(0, REM)]
    kvr = kvb_a.at[pl.ds(0, REM)]
    pltpu.async_copy(q_hbm.at[jc_d], qr, sem_a)
    pltpu.async_copy(kv_hbm.at[jc_s], kvr, sem_a)
    pltpu.make_async_copy(q_hbm.at[jc_d], qr, sem_a).wait()
    pltpu.make_async_copy(kv_hbm.at[jc_s], kvr, sem_a).wait()

    eidx = lax.iota(jnp.int32, 16)
    dstv = jr_d[...]
    jd4[...] = lax.shift_right_logical(dstv, 2)
    colb = lax.shift_left(jnp.bitwise_and(dstv, 3), 1)
    for col8 in range(8):
        plsc.store_scatter(dprod, [eidx, jnp.full((16,), col8, jnp.int32)], zero16)
    for h in range(2):
        alpha = jnp.zeros((16,), jnp.float32)
        for d in range(DD):
            cv16 = jnp.full((16,), h * DD + d, jnp.int32)
            qt = plsc.load_gather(qb_a, [eidx, cv16])
            kt = plsc.load_gather(kvb_a, [eidx, cv16])
            alpha = alpha + qt * kt
        ex = jnp.exp(alpha)
        plsc.store_scatter(dprod, [eidx, colb + h], ex)
        for d in range(DD):
            vt = plsc.load_gather(kvb_a, [eidx, jnp.full((16,), CH2 + h * DD + d, jnp.int32)])
            plsc.store_scatter(vprod, [eidx, jnp.full((16,), h * DD + d, jnp.int32)], vt * ex)
    pltpu.sync_copy(vprod.at[pl.ds(0, REM)], nacc.at[jr_d], add=True)
    pltpu.sync_copy(dprod.at[pl.ds(0, REM)], dacc.at[jd4], add=True)

    plsc.subcore_barrier()

    def writeout(nm_hbm, dn_hbm):
        @pl.when(t < NTILES - 1)
        def _():
            pltpu.sync_copy(nacc.at[pl.ds(r0, Z_A)], nm_hbm.at[pl.ds(r0, Z_A)])
            pltpu.sync_copy(dacc.at[pl.ds(s0, D_A)], dn_hbm.at[pl.ds(s0, D_A)])

        @pl.when(t == NTILES - 1)
        def _():
            pltpu.sync_copy(nacc.at[pl.ds(r0, Z_B)], nm_hbm.at[pl.ds(r0, Z_B)])
            pltpu.sync_copy(dacc.at[pl.ds(s0, D_B)], dn_hbm.at[pl.ds(s0, D_B)])

    @pl.when(c == 0)
    def _():
        writeout(nm0_hbm, dn0_hbm)

    @pl.when(c == 1)
    def _():
        writeout(nm1_hbm, dn1_hbm)


@functools.cache
def _build_sc_edge():
    mesh = plsc.VectorSubcoreMesh(core_axis_name="c", subcore_axis_name="s",
                                  num_cores=2, num_subcores=NTILES)
    return functools.partial(
        pl.kernel,
        out_type=[jax.ShapeDtypeStruct((NN, CH2), jnp.float32),
                  jax.ShapeDtypeStruct((NN, CH2), jnp.float32),
                  jax.ShapeDtypeStruct((ND, 8), jnp.float32),
                  jax.ShapeDtypeStruct((ND, 8), jnp.float32)],
        mesh=mesh,
        scratch_types=(
            [pltpu.VMEM((BKE,), jnp.int32)] * 2
            + [pltpu.VMEM((CH,), jnp.int32)] * 7
            + [pltpu.VMEM((REM,), jnp.int32)] * 4
            + [
                pltpu.VMEM((CH, CH2), jnp.float32),
                pltpu.VMEM((CH, CH2), jnp.float32),
                pltpu.VMEM((CH, CC), jnp.float32),
                pltpu.VMEM((CH, CC), jnp.float32),
                pltpu.VMEM((CH, CH2), jnp.float32),
                pltpu.VMEM((CH, 8), jnp.float32),
                pltpu.VMEM_SHARED((NN, CH2), jnp.float32),
                pltpu.VMEM_SHARED((ND, 8), jnp.float32),
                pltpu.SemaphoreType.DMA,
                pltpu.SemaphoreType.DMA,
            ]
        ),
        compiler_params=pltpu.CompilerParams(use_tc_tiling_on_sc=False,
                                             needs_layout_passes=False),
    )(_sc_edge_body)


def _sc_edge(qi, kvi, src, dst, zn, zd):
    return _build_sc_edge()(qi, kvi, src, dst, zn, zd)


def kernel(x_operator, edge_index_calledby, batch_operator, W_in, b_in, Wk, bk,
           Wq, bq, Wv, bv, a_rel, m_rel, p_rel, Wa, ba, skip, ln_g, ln_b, Wm,
           bm, Wt, bt):
    f32 = jnp.float32
    src = jnp.pad(edge_index_calledby[0], (0, BKE))
    dst = jnp.pad(edge_index_calledby[1], (0, BKE))

    colscale = jnp.repeat(p_rel / jnp.sqrt(jnp.float32(DD)), DD)
    Wq_s = Wq * colscale[None, :]
    bq_s = (bq * colscale).reshape(1, CC)
    A_blk = jax.scipy.linalg.block_diag(*a_rel)
    M_blk = jax.scipy.linalg.block_diag(*m_rel)
    WkA = Wk @ A_blk
    bkA = (bk @ A_blk).reshape(1, CC)
    WvM = Wv @ M_blk
    bvM = (bv @ M_blk).reshape(1, CC)
    S = jnp.repeat(jnp.eye(HH, dtype=f32), DD, axis=1)
    beta = jax.nn.sigmoid(skip).reshape(1, 1)
    Wmt = jnp.concatenate([Wm, Wt], axis=1)
    bmt = jnp.concatenate([bm, bt]).reshape(1, 2)
    bo = batch_operator.reshape(NN, 1)
    zn = jnp.zeros((Z_A, CH2), f32)
    zd = jnp.zeros((D_A, 8), f32)

    h, q, kvf = _proj_in(x_operator, W_in, b_in.reshape(1, CC),
                         Wq_s, bq_s, WkA, bkA, WvM, bvM)
    for layer in range(LL):
        nm0, nm1, dp0, dp1 = _sc_edge(q.reshape(2 * NN, CH2),
                                      kvf.reshape(2 * NN, CC),
                                      src, dst, zn, zd)
        dn = jnp.concatenate([dp0.reshape(NN, 2), dp1.reshape(NN, 2)], axis=1)
        if layer < LL - 1:
            h, q, kvf = _stage_b_proj(
                nm0, nm1, dn, h, S, Wa, ba.reshape(1, CC), beta,
                ln_g.reshape(1, CC), ln_b.reshape(1, CC),
                Wq_s, bq_s, WkA, bkA, WvM, bvM)
        else:
            h = _stage_b_only(nm0, nm1, dn, h, S, Wa, ba.reshape(1, CC), beta,
                              ln_g.reshape(1, CC), ln_b.reshape(1, CC))

    out2 = _pool(h, bo, Wmt, bmt)
    return out2[:, 0], out2[:, 1]
